# baseline (speedup 1.0000x reference)
import functools

import jax
import jax.numpy as jnp
from jax import lax
from jax.experimental import pallas as pl
from jax.experimental.pallas import tpu as pltpu
from jax.experimental.pallas import tpu_sc as plsc

VOCAB = 1000000
DIM = 64
LANES = 128
BATCH, SEQ = 16384, 50
TOTAL = BATCH * SEQ
NC, NS = 2, 16
NW = NC * NS
CHUNK = 128
N_GROUPS = TOTAL // CHUNK
GPW = N_GROUPS // NW
BBLK = BATCH // LANES
PAIRS = GPW // 2
DT = DIM // 8

_mesh = plsc.VectorSubcoreMesh(core_axis_name="c", subcore_axis_name="s")


@functools.partial(
    pl.kernel,
    mesh=_mesh,
    compiler_params=pltpu.CompilerParams(use_tc_tiling_on_sc=False,
                                         needs_layout_passes=False),
    out_type=jax.ShapeDtypeStruct((SEQ, DT, BBLK, 8, LANES), jnp.float32),
    scratch_types=[
        pltpu.VMEM((GPW, CHUNK), jnp.int32),
        pltpu.VMEM((CHUNK, DIM), jnp.float32),
        pltpu.VMEM((CHUNK, DIM), jnp.float32),
        pltpu.VMEM((DIM, LANES), jnp.float32),
        pltpu.VMEM((DIM, LANES), jnp.float32),
        pltpu.SemaphoreType.DMA,
        pltpu.SemaphoreType.DMA,
        pltpu.SemaphoreType.DMA,
        pltpu.SemaphoreType.DMA,
    ],
)
def _embed_sc(x_hbm, w_hbm, out_hbm, idx_v, g_a, g_b, t_a, t_b,
              gsem_a, gsem_b, wsem_a, wsem_b):
    wid = lax.axis_index("s") * NC + lax.axis_index("c")
    gbase = wid * GPW
    pltpu.sync_copy(x_hbm.at[wid], idx_v)

    lane = lax.iota(jnp.int32, 16)
    jrows = [lane + (jb * 16) for jb in range(CHUNK // 16)]

    def start_gather(k, gbuf, sem):
        pltpu.async_copy(w_hbm.at[idx_v.at[k]], gbuf, sem)

    def wait_gather(gbuf, sem):
        pltpu.make_async_copy(w_hbm.at[idx_v.at[0]], gbuf, sem).wait()

    def transpose(gbuf, tbuf):
        def drow(d, carry):
            dv = jnp.full((16,), d, jnp.int32)
            for jb in range(CHUNK // 16):
                tbuf[d, pl.ds(jb * 16, 16)] = plsc.load_gather(
                    gbuf, [jrows[jb], dv])
            return carry
        lax.fori_loop(0, DIM, drow, 0)

    def start_writes(k, tbuf, sem):
        g = gbase + k
        s = g // BBLK
        bt = g % BBLK
        for dt in range(DT):
            pltpu.async_copy(tbuf.at[pl.ds(dt * 8, 8)], out_hbm.at[s, dt, bt],
                             sem)

    def wait_writes(tbuf, sem):
        for dt in range(DT):
            pltpu.make_async_copy(tbuf.at[pl.ds(dt * 8, 8)],
                                  out_hbm.at[0, dt, 0], sem).wait()

    start_gather(0, g_a, gsem_a)

    def body(k, carry):
        ka = 2 * k
        kb = 2 * k + 1
        start_gather(kb, g_b, gsem_b)
        wait_gather(g_a, gsem_a)

        @pl.when(k > 0)
        def _():
            wait_writes(t_a, wsem_a)

        transpose(g_a, t_a)
        start_writes(ka, t_a, wsem_a)

        @pl.when(k < PAIRS - 1)
        def _():
            start_gather(ka + 2, g_a, gsem_a)

        wait_gather(g_b, gsem_b)

        @pl.when(k > 0)
        def _():
            wait_writes(t_b, wsem_b)

        transpose(g_b, t_b)
        start_writes(kb, t_b, wsem_b)
        return carry

    lax.fori_loop(0, PAIRS, body, 0)
    wait_writes(t_a, wsem_a)
    wait_writes(t_b, wsem_b)


def kernel(x, weight):
    xf = x.T.reshape(NW, GPW, CHUNK)
    out = _embed_sc(xf, weight)
    return jnp.transpose(out, (2, 4, 0, 1, 3)).reshape(BATCH, SEQ, DIM)


# speedup vs baseline: 1.1450x; 1.1450x over previous
import functools

import jax
import jax.numpy as jnp
from jax import lax
from jax.experimental import pallas as pl
from jax.experimental.pallas import tpu as pltpu
from jax.experimental.pallas import tpu_sc as plsc

VOCAB = 1000000
DIM = 64
LANES = 128
BATCH, SEQ = 16384, 50
TOTAL = BATCH * SEQ
NC, NS = 2, 16
NW = NC * NS
CHUNK = 128
N_GROUPS = TOTAL // CHUNK
GPW = N_GROUPS // NW
BBLK = BATCH // LANES
PAIRS = GPW // 2
DT = DIM // 8
TILE = 8 * LANES

_mesh = plsc.VectorSubcoreMesh(core_axis_name="c", subcore_axis_name="s")


@functools.partial(
    pl.kernel,
    mesh=_mesh,
    compiler_params=pltpu.CompilerParams(use_tc_tiling_on_sc=False,
                                         needs_layout_passes=False,
                                         disable_bounds_checks=True),
    out_type=jax.ShapeDtypeStruct((SEQ, DT, BBLK, TILE), jnp.float32),
    scratch_types=[
        pltpu.VMEM((GPW, CHUNK), jnp.int32),
        pltpu.VMEM((CHUNK, DIM), jnp.float32),
        pltpu.VMEM((CHUNK, DIM), jnp.float32),
        pltpu.VMEM((DIM * LANES,), jnp.float32),
        pltpu.VMEM((DIM * LANES,), jnp.float32),
        pltpu.SemaphoreType.DMA,
        pltpu.SemaphoreType.DMA,
        pltpu.SemaphoreType.DMA,
        pltpu.SemaphoreType.DMA,
    ],
)
def _embed_sc(x_hbm, w_hbm, out_hbm, idx_v, g_a, g_b, t_a, t_b,
              gsem_a, gsem_b, wsem_a, wsem_b):
    wid = lax.axis_index("s") * NC + lax.axis_index("c")
    gbase = wid * GPW
    pltpu.sync_copy(x_hbm.at[wid], idx_v)

    lane = lax.iota(jnp.int32, 16)
    dbases = [(lane + db * 16) * LANES for db in range(DIM // 16)]

    def start_gather(k, gbuf, sem):
        pltpu.async_copy(w_hbm.at[idx_v.at[k]], gbuf, sem)

    def wait_gather(gbuf, sem):
        pltpu.make_async_copy(w_hbm.at[idx_v.at[0]], gbuf, sem).wait()

    def transpose(gbuf, tbuf):
        def jrow(j, carry):
            for db in range(DIM // 16):
                v = gbuf[j, pl.ds(db * 16, 16)]
                plsc.store_scatter(tbuf, [dbases[db] + j], v)
            return carry
        lax.fori_loop(0, CHUNK, jrow, 0, unroll=4)

    def start_writes(k, tbuf, sem):
        g = gbase + k
        s = g // BBLK
        bt = g % BBLK
        for dt in range(DT):
            pltpu.async_copy(tbuf.at[pl.ds(dt * TILE, TILE)],
                             out_hbm.at[s, dt, bt], sem)

    def wait_writes(tbuf, sem):
        for dt in range(DT):
            pltpu.make_async_copy(tbuf.at[pl.ds(dt * TILE, TILE)],
                                  out_hbm.at[0, dt, 0], sem).wait()

    start_gather(0, g_a, gsem_a)

    def body(k, carry):
        ka = 2 * k
        kb = 2 * k + 1
        start_gather(kb, g_b, gsem_b)
        wait_gather(g_a, gsem_a)

        @pl.when(k > 0)
        def _():
            wait_writes(t_a, wsem_a)

        transpose(g_a, t_a)
        start_writes(ka, t_a, wsem_a)

        @pl.when(k < PAIRS - 1)
        def _():
            start_gather(ka + 2, g_a, gsem_a)

        wait_gather(g_b, gsem_b)

        @pl.when(k > 0)
        def _():
            wait_writes(t_b, wsem_b)

        transpose(g_b, t_b)
        start_writes(kb, t_b, wsem_b)
        return carry

    lax.fori_loop(0, PAIRS, body, 0)
    wait_writes(t_a, wsem_a)
    wait_writes(t_b, wsem_b)


def kernel(x, weight):
    xf = x.T.reshape(NW, GPW, CHUNK)
    out = _embed_sc(xf, weight)
    out5 = out.reshape(SEQ, DT, BBLK, 8, LANES)
    return jnp.transpose(out5, (2, 4, 0, 1, 3)).reshape(BATCH, SEQ, DIM)


# speedup vs baseline: 1.9524x; 1.7051x over previous
import functools

import jax
import jax.numpy as jnp
from jax import lax
from jax.experimental import pallas as pl
from jax.experimental.pallas import tpu as pltpu
from jax.experimental.pallas import tpu_sc as plsc

VOCAB = 1000000
DIM = 64
BATCH, SEQ = 16384, 50
SEQP, LANES = 56, 128
TOTAL = BATCH * SEQ
NC, NS = 2, 16
NW = NC * NS
B_PER_W = BATCH // NW
CHUNK = 100
N_CHUNKS = B_PER_W * SEQ // CHUNK
GPC = 4
GROUP = CHUNK * GPC
BPG = GROUP // SEQ
PAIRS = N_CHUNKS // (2 * GPC)

_mesh = plsc.VectorSubcoreMesh(core_axis_name="c", subcore_axis_name="s")


@functools.partial(
    pl.kernel,
    mesh=_mesh,
    compiler_params=pltpu.CompilerParams(use_tc_tiling_on_sc=False,
                                         needs_layout_passes=False,
                                         disable_bounds_checks=True),
    out_type=jax.ShapeDtypeStruct((BATCH, SEQP, LANES), jnp.float32),
    scratch_types=[
        pltpu.VMEM((N_CHUNKS, CHUNK), jnp.int32),
        pltpu.VMEM((GROUP, DIM), jnp.float32),
        pltpu.VMEM((GROUP, DIM), jnp.float32),
        pltpu.SemaphoreType.DMA,
        pltpu.SemaphoreType.DMA,
        pltpu.SemaphoreType.DMA,
        pltpu.SemaphoreType.DMA,
    ],
)
def _embed_sc(x_hbm, w_hbm, out_hbm, idx_v, buf_a, buf_b, gsem_a, gsem_b,
              wsem_a, wsem_b):
    wid = lax.axis_index("s") * NC + lax.axis_index("c")
    bbase = wid * B_PER_W
    pltpu.sync_copy(x_hbm.at[wid], idx_v)

    def start_gathers(group, buf, sem):
        for c in range(GPC):
            pltpu.async_copy(w_hbm.at[idx_v.at[group * GPC + c]],
                             buf.at[pl.ds(c * CHUNK, CHUNK)], sem)

    def wait_gathers(buf, sem):
        for c in range(GPC):
            pltpu.make_async_copy(w_hbm.at[idx_v.at[0]],
                                  buf.at[pl.ds(c * CHUNK, CHUNK)], sem).wait()

    def start_writes(group, buf, sem):
        for i in range(BPG):
            pltpu.async_copy(
                buf.at[pl.ds(i * SEQ, SEQ)],
                out_hbm.at[bbase + group * BPG + i, pl.ds(0, SEQ),
                           pl.ds(0, DIM)],
                sem)

    def wait_writes(buf, sem):
        for i in range(BPG):
            pltpu.make_async_copy(
                buf.at[pl.ds(i * SEQ, SEQ)],
                out_hbm.at[0, pl.ds(0, SEQ), pl.ds(0, DIM)], sem).wait()

    start_gathers(0, buf_a, gsem_a)

    def body(k, carry):
        wait_gathers(buf_a, gsem_a)
        start_writes(2 * k, buf_a, wsem_a)

        @pl.when(k > 0)
        def _():
            wait_writes(buf_b, wsem_b)

        start_gathers(2 * k + 1, buf_b, gsem_b)

        wait_writes(buf_a, wsem_a)

        @pl.when(k < PAIRS - 1)
        def _():
            start_gathers(2 * k + 2, buf_a, gsem_a)

        wait_gathers(buf_b, gsem_b)
        start_writes(2 * k + 1, buf_b, wsem_b)
        return carry

    lax.fori_loop(0, PAIRS, body, 0)
    wait_writes(buf_b, wsem_b)


def kernel(x, weight):
    xf = x.reshape(NW, N_CHUNKS, CHUNK)
    out = _embed_sc(xf, weight)
    return out[:, :SEQ, :DIM]
